# Initial kernel scaffold; baseline (speedup 1.0000x reference)
import jax
import jax.numpy as jnp
from jax.experimental import pallas as pl


def kernel(x, edge_index, W1, b1, W2, b2, CW1, Cb1, CW2, Cb2):
    raise NotImplementedError("write your pallas kernel here")


# speedup vs baseline: 41.6288x; 41.6288x over previous
import functools

import jax
import jax.numpy as jnp
from jax import lax
from jax.experimental import pallas as pl
from jax.experimental.pallas import tpu as pltpu
from jax.experimental.pallas import tpu_sc as plsc

N_NODES = 100000
N_EDGES = 3200000

NPAD = 100352
ROWS8 = NPAD // 8
NC, NS = 2, 16
NW = NC * NS
EPT = 102400
EPAD = EPT * NW
IDX_ROWS = EPAD // 128
TROWS = EPT // 128
KG = 8
GROUPS = TROWS // KG
KD = 16
DGROUPS = TROWS // KD
SLICE = NPAD // NS


def _deg_body(dst_hbm, zeros1_hbm, out_hbm, acc, didx, ones):
    c = lax.axis_index("c")
    s = lax.axis_index("s")
    w = c * NS + s
    pltpu.sync_copy(zeros1_hbm.at[pl.ds(s * SLICE, SLICE)],
                    acc.at[pl.ds(s * SLICE, SLICE)])
    for i in range(8):
        ones[pl.ds(i * 16, 16)] = jnp.full((16,), 1.0, jnp.float32)
    plsc.subcore_barrier()
    base = w * TROWS

    def body(g, carry):
        row = base + g * KD
        pltpu.sync_copy(dst_hbm.at[pl.ds(row, KD)], didx)
        for j in range(KD):
            pltpu.sync_copy(ones, acc.at[didx.at[j]], add=True)
        return carry

    lax.fori_loop(0, DGROUPS, body, 0)
    plsc.subcore_barrier()
    pltpu.sync_copy(acc.at[pl.ds(s * SLICE, SLICE)],
                    out_hbm.at[pl.ds(c * NPAD + s * SLICE, SLICE)])


@functools.cache
def _deg_call():
    mesh = plsc.VectorSubcoreMesh(core_axis_name="c", subcore_axis_name="s",
                                  num_cores=NC, num_subcores=NS)
    return pl.kernel(
        _deg_body,
        out_type=jax.ShapeDtypeStruct((NC * NPAD,), jnp.float32),
        mesh=mesh,
        compiler_params=pltpu.CompilerParams(use_tc_tiling_on_sc=False),
        scratch_types=[
            pltpu.VMEM_SHARED((NPAD,), jnp.float32),
            pltpu.VMEM((KD, 128), jnp.int32),
            pltpu.VMEM((128,), jnp.float32),
        ],
    )


def _agg_body(src_hbm, dst_hbm, table_hbm, zeros2_hbm, out_hbm,
              acc, sidx, didx, rows, sem):
    c = lax.axis_index("c")
    s = lax.axis_index("s")
    w = c * NS + s
    pltpu.sync_copy(zeros2_hbm.at[pl.ds(s * SLICE, SLICE)],
                    acc.at[pl.ds(s * SLICE, SLICE)])
    plsc.subcore_barrier()
    base = w * TROWS

    def body(g, carry):
        row = base + g * KG
        pltpu.sync_copy(src_hbm.at[pl.ds(row, KG)], sidx)
        pltpu.sync_copy(dst_hbm.at[pl.ds(row, KG)], didx)
        descs = [pltpu.async_copy(table_hbm.at[sidx.at[j]], rows.at[j], sem)
                 for j in range(KG)]
        for d in descs:
            d.wait()
        for j in range(KG):
            pltpu.sync_copy(rows.at[j], acc.at[didx.at[j]], add=True)
        return carry

    lax.fori_loop(0, GROUPS, body, 0)
    plsc.subcore_barrier()
    pltpu.sync_copy(acc.at[pl.ds(s * SLICE, SLICE)],
                    out_hbm.at[pl.ds(c * NPAD + s * SLICE, SLICE)])


@functools.cache
def _agg_call():
    mesh = plsc.VectorSubcoreMesh(core_axis_name="c", subcore_axis_name="s",
                                  num_cores=NC, num_subcores=NS)
    return pl.kernel(
        _agg_body,
        out_type=jax.ShapeDtypeStruct((NC * NPAD, 16), jnp.float32),
        mesh=mesh,
        compiler_params=pltpu.CompilerParams(use_tc_tiling_on_sc=False),
        scratch_types=[
            pltpu.VMEM_SHARED((NPAD, 16), jnp.float32),
            pltpu.VMEM((KG, 128), jnp.int32),
            pltpu.VMEM((KG, 128), jnp.int32),
            pltpu.VMEM((KG, 128, 16), jnp.float32),
            pltpu.SemaphoreType.DMA,
        ],
    )


def _tc_a(p_ref, x48_ref, w1bd_ref, e8_ref, dinv_ref, hs1_ref):
    deg8 = p_ref[0] + p_ref[1] + 1.0
    dinv8 = lax.rsqrt(deg8)
    dinv = jnp.dot(dinv8, e8_ref[...], preferred_element_type=jnp.float32)
    dinv_ref[...] = dinv
    xw = jnp.dot(x48_ref[...], w1bd_ref[...], preferred_element_type=jnp.float32)
    hs1_ref[...] = xw * dinv


def _tc_b(q_ref, hs1_ref, dinv_ref, b1t_ref, w2bd_ref, hs2_ref):
    dinv = dinv_ref[...]
    h1 = jnp.maximum(dinv * (q_ref[0] + q_ref[1] + hs1_ref[...]) + b1t_ref[...],
                     0.0)
    hs2_ref[...] = jnp.dot(h1, w2bd_ref[...],
                           preferred_element_type=jnp.float32) * dinv


def _tc_c(r_ref, hs2_ref, dinv_ref, b2t_ref, cw1bd_ref, cw2bd_ref,
          cb1t_ref, cb2t_ref, o1_ref, o2_ref):
    dinv = dinv_ref[...]
    h2 = jnp.maximum(dinv * (r_ref[0] + r_ref[1] + hs2_ref[...]) + b2t_ref[...],
                     0.0)
    o1_ref[...] = jnp.dot(h2, cw1bd_ref[...],
                          preferred_element_type=jnp.float32) + cb1t_ref[...]
    o2_ref[...] = jnp.dot(h2, cw2bd_ref[...],
                          preferred_element_type=jnp.float32) + cb2t_ref[...]


_tc_a_call = pl.pallas_call(
    _tc_a,
    out_shape=[jax.ShapeDtypeStruct((ROWS8, 128), jnp.float32),
               jax.ShapeDtypeStruct((ROWS8, 128), jnp.float32)],
)

_tc_b_call = pl.pallas_call(
    _tc_b,
    out_shape=jax.ShapeDtypeStruct((ROWS8, 128), jnp.float32),
)

_tc_c_call = pl.pallas_call(
    _tc_c,
    out_shape=[jax.ShapeDtypeStruct((ROWS8, 104), jnp.float32),
               jax.ShapeDtypeStruct((ROWS8, 64), jnp.float32)],
)


def kernel(x, edge_index, W1, b1, W2, b2, CW1, Cb1, CW2, Cb2):
    src = edge_index[0]
    dst = edge_index[1]
    padv = jnp.full((EPAD - N_EDGES,), N_NODES, jnp.int32)
    src_p = jnp.concatenate([src, padv]).reshape(IDX_ROWS, 128)
    dst_p = jnp.concatenate([dst, padv]).reshape(IDX_ROWS, 128)
    zeros1 = jnp.zeros((NPAD,), jnp.float32)
    zeros2 = jnp.zeros((NPAD, 16), jnp.float32)

    x48 = jnp.pad(x, ((0, NPAD - N_NODES), (0, 0))).reshape(ROWS8, 48)
    eye8 = jnp.eye(8, dtype=jnp.float32)
    w1bd = jnp.kron(eye8, W1)
    w2bd = jnp.kron(eye8, W2)
    cw1bd = jnp.kron(eye8, CW1)
    cw2bd = jnp.kron(eye8, CW2)
    b1t = jnp.tile(b1, 8)[None, :]
    b2t = jnp.tile(b2, 8)[None, :]
    cb1t = jnp.tile(Cb1, 8)[None, :]
    cb2t = jnp.tile(Cb2, 8)[None, :]
    e8 = jnp.repeat(eye8, 16, axis=1)

    degp = _deg_call()(dst_p, zeros1).reshape(NC, ROWS8, 8)
    dinv, hs1 = _tc_a_call(degp, x48, w1bd, e8)
    q = _agg_call()(src_p, dst_p, hs1.reshape(NPAD, 16), zeros2)
    hs2 = _tc_b_call(q.reshape(NC, ROWS8, 128), hs1, dinv, b1t, w2bd)
    r = _agg_call()(src_p, dst_p, hs2.reshape(NPAD, 16), zeros2)
    o1p, o2p = _tc_c_call(r.reshape(NC, ROWS8, 128), hs2, dinv, b2t,
                          cw1bd, cw2bd, cb1t, cb2t)
    out_1 = o1p.reshape(NPAD, 13)[:N_NODES]
    out_2 = o2p.reshape(NPAD, 8)[:N_NODES]
    return (out_1, out_2)
